# baseline (speedup 1.0000x reference)
import jax
import jax.numpy as jnp
from jax.experimental import pallas as pl
from jax.experimental.pallas import tpu as pltpu


def _masked_copy_kernel(x_ref, o_ref, zeros_vmem, sem_zero, sem_copy):
    nb = x_ref.shape[0]
    copy_rest = pltpu.make_async_copy(
        x_ref.at[pl.ds(1, nb - 1)], o_ref.at[pl.ds(1, nb - 1)], sem_copy
    )
    copy_rest.start()
    zeros_vmem[...] = jnp.zeros_like(zeros_vmem)
    copy_zero = pltpu.make_async_copy(zeros_vmem, o_ref.at[0], sem_zero)
    copy_zero.start()
    copy_rest.wait()
    copy_zero.wait()


def kernel(time_images_season_list):
    x = time_images_season_list
    _, b, t, c, n = x.shape
    x2 = x.reshape(b, t, c * n)
    out = pl.pallas_call(
        _masked_copy_kernel,
        in_specs=[pl.BlockSpec(memory_space=pl.ANY)],
        out_specs=pl.BlockSpec(memory_space=pl.ANY),
        out_shape=jax.ShapeDtypeStruct((b, t, c * n), x.dtype),
        scratch_shapes=[
            pltpu.VMEM((t, c * n), x.dtype),
            pltpu.SemaphoreType.DMA,
            pltpu.SemaphoreType.DMA,
        ],
    )(x2)
    return out.reshape(b, t, c, n)
